# Initial kernel scaffold; baseline (speedup 1.0000x reference)
import jax
import jax.numpy as jnp
from jax.experimental import pallas as pl


def kernel(indices, table):
    raise NotImplementedError("write your pallas kernel here")


# speedup vs baseline: 3.1565x; 3.1565x over previous
import functools

import jax
import jax.numpy as jnp
from jax import lax
from jax.experimental import pallas as pl
from jax.experimental.pallas import tpu as pltpu
from jax.experimental.pallas import tpu_sc as plsc

NC = 2
NS = 16
NW = NC * NS
B = 204800
D = 64
BPW = B // NW
CHUNK = 128
NCHUNK = BPW // CHUNK

_mesh = plsc.VectorSubcoreMesh(core_axis_name="c", subcore_axis_name="s")


@functools.partial(
    pl.kernel,
    mesh=_mesh,
    out_type=jax.ShapeDtypeStruct((B, D), jnp.float32),
    scratch_types=[
        pltpu.VMEM((NCHUNK, CHUNK), jnp.int32),
        pltpu.VMEM((CHUNK, D), jnp.float32),
        pltpu.SemaphoreType.DMA,
    ],
    compiler_params=pltpu.CompilerParams(use_tc_tiling_on_sc=False),
)
def _gather_kernel(idx_hbm, table_hbm, out_hbm, idx_v, rows_v, sem):
    wid = lax.axis_index("s") * NC + lax.axis_index("c")
    base = wid * BPW
    pltpu.sync_copy(idx_hbm.at[wid], idx_v)

    def body(j, carry):
        pltpu.async_copy(table_hbm.at[idx_v.at[j]], rows_v, sem).wait()
        pltpu.sync_copy(rows_v, out_hbm.at[pl.ds(base + j * CHUNK, CHUNK)])
        return carry

    lax.fori_loop(0, NCHUNK, body, 0)


def kernel(indices, table):
    idx = indices.astype(jnp.int32).reshape(NW, NCHUNK, CHUNK)
    return _gather_kernel(idx, table)


# speedup vs baseline: 3.5944x; 1.1387x over previous
import functools

import jax
import jax.numpy as jnp
from jax import lax
from jax.experimental import pallas as pl
from jax.experimental.pallas import tpu as pltpu
from jax.experimental.pallas import tpu_sc as plsc

NC = 2
NS = 16
NW = NC * NS
B = 204800
D = 64
BPW = B // NW
CHUNK = 128
NCHUNK = BPW // CHUNK
GROUP = 640
NDMA = GROUP // CHUNK
NGROUP = BPW // GROUP

_mesh = plsc.VectorSubcoreMesh(core_axis_name="c", subcore_axis_name="s")


@functools.partial(
    pl.kernel,
    mesh=_mesh,
    out_type=jax.ShapeDtypeStruct((B, D), jnp.float32),
    scratch_types=[
        pltpu.VMEM((NCHUNK, CHUNK), jnp.int32),
        pltpu.VMEM((2, GROUP, D), jnp.float32),
        pltpu.SemaphoreType.DMA((2,)),
    ],
    compiler_params=pltpu.CompilerParams(use_tc_tiling_on_sc=False),
)
def _gather_kernel(idx_hbm, table_hbm, out_hbm, idx_v, rows_v, gsem):
    wid = lax.axis_index("s") * NC + lax.axis_index("c")
    base = wid * BPW
    pltpu.sync_copy(idx_hbm.at[wid], idx_v)

    def fire(g, buf):
        for i in range(NDMA):
            pltpu.async_copy(
                table_hbm.at[idx_v.at[g * NDMA + i]],
                rows_v.at[buf].at[pl.ds(i * CHUNK, CHUNK)],
                gsem.at[buf],
            )

    def drain(g, buf):
        for i in range(NDMA):
            pltpu.make_async_copy(
                table_hbm.at[idx_v.at[g * NDMA + i]],
                rows_v.at[buf].at[pl.ds(i * CHUNK, CHUNK)],
                gsem.at[buf],
            ).wait()

    fire(0, 0)
    fire(1, 1)
    for g in range(NGROUP):
        buf = g % 2
        drain(g, buf)
        pltpu.sync_copy(rows_v.at[buf], out_hbm.at[pl.ds(base + g * GROUP, GROUP)])
        if g + 2 < NGROUP:
            fire(g + 2, buf)


def kernel(indices, table):
    idx = indices.astype(jnp.int32).reshape(NW, NCHUNK, CHUNK)
    return _gather_kernel(idx, table)
